# Initial kernel scaffold; baseline (speedup 1.0000x reference)
import jax
import jax.numpy as jnp
from jax.experimental import pallas as pl
from jax.experimental.pallas import tpu as pltpu

_LANES = 4096
_ROWS = 64 * 128 * 256 * 4 * 4 // _LANES
_BLOCK_ROWS = 512


def _swap_body(x_ref, o_ref):
    v = x_ref[...]
    lane = jax.lax.broadcasted_iota(jnp.int32, v.shape, 1)
    fwd = pltpu.roll(v, -2, 1)
    bwd = pltpu.roll(v, 2, 1)
    o_ref[...] = jnp.where((lane & 2) == 0, fwd, bwd)


def kernel(x):
    orig_shape = x.shape
    x2 = x.reshape(_ROWS, _LANES)
    out = pl.pallas_call(
        _swap_body,
        grid=(_ROWS // _BLOCK_ROWS,),
        in_specs=[pl.BlockSpec((_BLOCK_ROWS, _LANES), lambda i: (i, 0))],
        out_specs=pl.BlockSpec((_BLOCK_ROWS, _LANES), lambda i: (i, 0)),
        out_shape=jax.ShapeDtypeStruct((_ROWS, _LANES), x.dtype),
    )(x2)
    return out.reshape(orig_shape)


# speedup vs baseline: 4.8332x; 4.8332x over previous
import jax
import jax.numpy as jnp
from jax.experimental import pallas as pl
from jax.experimental.pallas import tpu as pltpu

_LANES = 4096
_ROWS = 64 * 128 * 256 * 4 * 4 // _LANES
_BLOCK_ROWS = 512


def _swap_body(x_ref, o_ref):
    v = x_ref[...]
    lane = jax.lax.broadcasted_iota(jnp.int32, v.shape, 1)
    fwd = pltpu.roll(v, v.shape[1] - 2, 1)
    bwd = pltpu.roll(v, 2, 1)
    o_ref[...] = jnp.where((lane & 2) == 0, fwd, bwd)


def kernel(x):
    orig_shape = x.shape
    x2 = x.reshape(_ROWS, _LANES)
    out = pl.pallas_call(
        _swap_body,
        grid=(_ROWS // _BLOCK_ROWS,),
        in_specs=[pl.BlockSpec((_BLOCK_ROWS, _LANES), lambda i: (i, 0))],
        out_specs=pl.BlockSpec((_BLOCK_ROWS, _LANES), lambda i: (i, 0)),
        out_shape=jax.ShapeDtypeStruct((_ROWS, _LANES), x.dtype),
    )(x2)
    return out.reshape(orig_shape)


# speedup vs baseline: 19.4298x; 4.0201x over previous
import jax
import jax.numpy as jnp
from jax.experimental import pallas as pl
from jax.experimental.pallas import tpu as pltpu

_ROWS = 64 * 128 * 4 * 2 * 4
_LANES = 128
_BLOCK_ROWS = 1024


def _swap_body(x_ref, o_ref):
    v = x_ref[...]
    sub = jax.lax.broadcasted_iota(jnp.int32, v.shape, 0)
    fwd = pltpu.roll(v, v.shape[0] - 2, 0)
    bwd = pltpu.roll(v, 2, 0)
    o_ref[...] = jnp.where((sub & 2) == 0, fwd, bwd)


def kernel(x):
    a, b, c, s, t = x.shape
    g, l = c // _LANES, _LANES
    xr = (
        x.transpose(0, 1, 3, 4, 2)
        .reshape(a, b, s, t, g, l)
        .transpose(0, 1, 2, 4, 3, 5)
        .reshape(_ROWS, _LANES)
    )
    out = pl.pallas_call(
        _swap_body,
        grid=(_ROWS // _BLOCK_ROWS,),
        in_specs=[pl.BlockSpec((_BLOCK_ROWS, _LANES), lambda i: (i, 0))],
        out_specs=pl.BlockSpec((_BLOCK_ROWS, _LANES), lambda i: (i, 0)),
        out_shape=jax.ShapeDtypeStruct((_ROWS, _LANES), x.dtype),
    )(xr)
    return (
        out.reshape(a, b, s, g, t, l)
        .transpose(0, 1, 2, 4, 3, 5)
        .reshape(a, b, s, t, c)
        .transpose(0, 1, 4, 2, 3)
    )


# speedup vs baseline: 43.3405x; 2.2306x over previous
import jax
import jax.numpy as jnp
from jax.experimental import pallas as pl
from jax.experimental.pallas import tpu as pltpu

_ROWS = 64 * 128 * 4 * 2 * 4
_LANES = 128
_BLOCK_ROWS = 8192


def _swap_body(x_ref, o_ref):
    v = x_ref[...]
    sub = jax.lax.broadcasted_iota(jnp.int32, v.shape, 0)
    fwd = pltpu.roll(v, v.shape[0] - 2, 0)
    bwd = pltpu.roll(v, 2, 0)
    o_ref[...] = jnp.where((sub & 2) == 0, fwd, bwd)


def kernel(x):
    a, b, c, s, t = x.shape
    g, l = c // _LANES, _LANES
    xr = (
        x.transpose(0, 1, 3, 4, 2)
        .reshape(a, b, s, t, g, l)
        .transpose(0, 1, 2, 4, 3, 5)
        .reshape(_ROWS, _LANES)
    )
    out = pl.pallas_call(
        _swap_body,
        grid=(_ROWS // _BLOCK_ROWS,),
        in_specs=[pl.BlockSpec((_BLOCK_ROWS, _LANES), lambda i: (i, 0))],
        out_specs=pl.BlockSpec((_BLOCK_ROWS, _LANES), lambda i: (i, 0)),
        out_shape=jax.ShapeDtypeStruct((_ROWS, _LANES), x.dtype),
    )(xr)
    return (
        out.reshape(a, b, s, g, t, l)
        .transpose(0, 1, 2, 4, 3, 5)
        .reshape(a, b, s, t, c)
        .transpose(0, 1, 4, 2, 3)
    )


# speedup vs baseline: 44.3910x; 1.0242x over previous
import jax
import jax.numpy as jnp
from jax.experimental import pallas as pl
from jax.experimental.pallas import tpu as pltpu

_ROWS = 64 * 128 * 4 * 2 * 4
_LANES = 128
_BLOCK_ROWS = 16384


def _swap_body(x_ref, o_ref):
    v = x_ref[...]
    sub = jax.lax.broadcasted_iota(jnp.int32, v.shape, 0)
    fwd = pltpu.roll(v, v.shape[0] - 2, 0)
    bwd = pltpu.roll(v, 2, 0)
    o_ref[...] = jnp.where((sub & 2) == 0, fwd, bwd)


def kernel(x):
    a, b, c, s, t = x.shape
    g, l = c // _LANES, _LANES
    xr = (
        x.transpose(0, 1, 3, 4, 2)
        .reshape(a, b, s, t, g, l)
        .transpose(0, 1, 2, 4, 3, 5)
        .reshape(_ROWS, _LANES)
    )
    out = pl.pallas_call(
        _swap_body,
        grid=(_ROWS // _BLOCK_ROWS,),
        in_specs=[pl.BlockSpec((_BLOCK_ROWS, _LANES), lambda i: (i, 0))],
        out_specs=pl.BlockSpec((_BLOCK_ROWS, _LANES), lambda i: (i, 0)),
        out_shape=jax.ShapeDtypeStruct((_ROWS, _LANES), x.dtype),
    )(xr)
    return (
        out.reshape(a, b, s, g, t, l)
        .transpose(0, 1, 2, 4, 3, 5)
        .reshape(a, b, s, t, c)
        .transpose(0, 1, 4, 2, 3)
    )
